# baseline (speedup 1.0000x reference)
import jax
import jax.numpy as jnp
from jax import lax
from jax.experimental import pallas as pl
from jax.experimental.pallas import tpu as pltpu
from jax.experimental.pallas import tpu_sc as plsc

_NC = 2
_NS = 16
_B = 128
_RPC = 32
_GB = 2
_H = 128
_S = 32


def _seg_sum_sc(xs, src, dst, n_dst):
    e = src.shape[0]
    ept = ((e + _NS * _RPC * _B - 1) // (_NS * _RPC * _B)) * _RPC * _B
    e_pad = ept * _NS
    rows_pt = ept // _B
    chunks = rows_pt // _RPC
    half_rows_pt = rows_pt // 2
    half_chunks = chunks // 2
    n_dst_pad = ((n_dst + 1 + 63) // 64) * 64
    zc = n_dst_pad // 32
    z_iter = (zc + _NS - 1) // _NS

    if e_pad > e:
        pad = e_pad - e
        src = jnp.concatenate([src, jnp.zeros((pad,), jnp.int32)])
        dst = jnp.concatenate([dst, jnp.full((pad,), jnp.int32(n_dst))])
    src2 = src.reshape(e_pad // _B, _B)
    dst2 = dst.reshape(e_pad // _B, _B)

    mesh = plsc.VectorSubcoreMesh(core_axis_name="c", subcore_axis_name="s")

    def body(x0, x1, x2, x3, src_hbm, dst_hbm, o0, o1, o2, o3, c0, c1,
             src_st, dst_st, rows, ones_blk, zrow, acc, sem, sem2):
        cid = lax.axis_index("c")
        sid = lax.axis_index("s")
        lane = lax.iota(jnp.int32, 16)
        onev = (1 - jnp.minimum(lane, 1)).astype(jnp.float32)

        def _zr(r, carry):
            for q in range(_S // 16):
                zrow[r, pl.ds(q * 16, 16)] = jnp.zeros((16,), jnp.float32)
            return carry
        lax.fori_loop(0, 32, _zr, 0)

        def _ob(r, carry):
            ones_blk[r, pl.ds(0, 16)] = onev
            ones_blk[r, pl.ds(16, 16)] = jnp.zeros((16,), jnp.float32)
            return carry
        lax.fori_loop(0, _B, _ob, 0)

        def zero_acc():
            def zb(z, carry):
                czk = z * _NS + sid
                @pl.when(czk < zc)
                def _():
                    pltpu.sync_copy(zrow, acc.at[pl.ds(czk * 32, 32), :])
                return carry
            lax.fori_loop(0, z_iter, zb, 0)

        def write_out(o):
            def wb(w, carry):
                wck = w * _NS + sid
                @pl.when(wck < zc)
                def _():
                    pltpu.sync_copy(acc.at[pl.ds(wck * 32, 32), :],
                                    o.at[pl.ds(wck * 32, 32), :])
                return carry
            lax.fori_loop(0, z_iter, wb, 0)

        def scan_pass(x, row0, nchunks):
            ngroups = _RPC // _GB

            def gather_descs(gi, par):
                return [pltpu.make_async_copy(
                            x.at[src_st.at[gi * _GB + b]],
                            rows.at[par, b], sem)
                        for b in range(_GB)]

            def chunk_body(ch, carry):
                base = row0 + ch * _RPC
                pltpu.sync_copy(src_hbm.at[pl.ds(base, _RPC), :], src_st)
                pltpu.sync_copy(dst_hbm.at[pl.ds(base, _RPC), :], dst_st)
                for d in gather_descs(0, 0):
                    d.start()

                def scat_descs(gi, par):
                    return [pltpu.make_async_copy(
                                rows.at[par, b],
                                acc.at[dst_st.at[gi * _GB + b]], sem2)
                            for b in range(_GB)]

                def grp(g, carry2):
                    par = g & 1
                    for d in gather_descs(g, par):
                        d.wait()
                    for b in range(_GB):
                        pltpu.async_copy(rows.at[par, b],
                                         acc.at[dst_st.at[g * _GB + b]],
                                         sem2, add=True)
                    @pl.when(g > 0)
                    def _():
                        for d in scat_descs(g - 1, 1 - par):
                            d.wait()
                    @pl.when(g + 1 < ngroups)
                    def _():
                        for d in gather_descs(g + 1, 1 - par):
                            d.start()
                    return carry2
                lax.fori_loop(0, ngroups, grp, 0)
                for d in scat_descs(ngroups - 1, (ngroups - 1) & 1):
                    d.wait()
                return carry
            lax.fori_loop(0, nchunks, chunk_body, 0)

        def count_pass(row0, nchunks):
            def chunk_body(ch, carry):
                base = row0 + ch * _RPC
                pltpu.sync_copy(dst_hbm.at[pl.ds(base, _RPC), :], dst_st)

                def one_row(j, carry2):
                    pltpu.async_copy(ones_blk, acc.at[dst_st.at[j]], sem2,
                                     add=True)
                    return carry2
                lax.fori_loop(0, _RPC, one_row, 0)

                def drain(j, carry2):
                    pltpu.make_async_copy(ones_blk,
                                          acc.at[dst_st.at[j]], sem2).wait()
                    return carry2
                lax.fori_loop(0, _RPC, drain, 0)
                return carry
            lax.fori_loop(0, nchunks, chunk_body, 0)

        xs_refs = (x0, x1, x2, x3)
        os_refs = (o0, o1, o2, o3)
        for p in range(4):
            own = cid == (p // 2)
            @pl.when(own)
            def _():
                zero_acc()
            plsc.subcore_barrier()
            @pl.when(own)
            def _():
                scan_pass(xs_refs[p], sid * rows_pt, chunks)
            plsc.subcore_barrier()
            @pl.when(own)
            def _():
                write_out(os_refs[p])
            plsc.subcore_barrier()

        zero_acc()
        plsc.subcore_barrier()
        count_pass(cid * (e_pad // _B // 2) + sid * half_rows_pt, half_chunks)
        plsc.subcore_barrier()
        @pl.when(cid == 0)
        def _():
            write_out(c0)
        @pl.when(cid == 1)
        def _():
            write_out(c1)

    kern = pl.kernel(
        body,
        out_type=tuple(jax.ShapeDtypeStruct((n_dst_pad, _S), jnp.float32)
                       for _ in range(6)),
        mesh=mesh,
        compiler_params=pltpu.CompilerParams(use_tc_tiling_on_sc=False),
        scratch_types=[
            pltpu.VMEM((_RPC, _B), jnp.int32),
            pltpu.VMEM((_RPC, _B), jnp.int32),
            pltpu.VMEM((2, _GB, _B, _S), jnp.float32),
            pltpu.VMEM((_B, _S), jnp.float32),
            pltpu.VMEM((32, _S), jnp.float32),
            pltpu.VMEM_SHARED((n_dst_pad, _S), jnp.float32),
            pltpu.SemaphoreType.DMA,
            pltpu.SemaphoreType.DMA,
        ],
    )
    return kern(xs[0], xs[1], xs[2], xs[3], src2, dst2)


_BLK = 2000


def _shard_specs(n):
    return [pl.BlockSpec((_BLK, _S), lambda i: (i, 0)) for _ in range(n)]


def _shard_outs(n):
    return tuple(jax.ShapeDtypeStruct((n, _S), jnp.float32) for _ in range(4))


def _enc2(x, w, b, w2):
    n = x.shape[0]

    def body(x_ref, w_ref, b_ref, w2_ref, o0, o1, o2, o3):
        h = jnp.maximum(jnp.dot(x_ref[...], w_ref[...],
                                preferred_element_type=jnp.float32)
                        + b_ref[...], 0.0)
        y = jnp.dot(h, w2_ref[...], preferred_element_type=jnp.float32)
        for k, o in enumerate((o0, o1, o2, o3)):
            o[...] = y[:, k * _S:(k + 1) * _S]

    return pl.pallas_call(
        body,
        grid=(n // _BLK,),
        in_specs=[pl.BlockSpec((_BLK, _H), lambda i: (i, 0)),
                  pl.BlockSpec((_H, _H), lambda i: (0, 0)),
                  pl.BlockSpec((1, _H), lambda i: (0, 0)),
                  pl.BlockSpec((_H, _H), lambda i: (0, 0))],
        out_specs=_shard_specs(4),
        out_shape=_shard_outs(n),
    )(x, w, b.reshape(1, _H), w2)


def _enc(x, w, b):
    n = x.shape[0]

    def body(x_ref, w_ref, b_ref, o_ref):
        o_ref[...] = jnp.maximum(
            jnp.dot(x_ref[...], w_ref[...], preferred_element_type=jnp.float32)
            + b_ref[...], 0.0)

    return pl.pallas_call(
        body,
        grid=(n // _BLK,),
        in_specs=[pl.BlockSpec((_BLK, _H), lambda i: (i, 0)),
                  pl.BlockSpec((_H, _H), lambda i: (0, 0)),
                  pl.BlockSpec((1, _H), lambda i: (0, 0))],
        out_specs=pl.BlockSpec((_BLK, _H), lambda i: (i, 0)),
        out_shape=jax.ShapeDtypeStruct((n, _H), jnp.float32),
    )(x, w, b.reshape(1, _H))


def _evt_update(evt_h, s4, c0, c1, wr, bl, wl_next):
    n = evt_h.shape[0]

    def body(e_ref, s0, s1, s2, s3, c0_ref, c1_ref, wr_ref, bl_ref, wn_ref,
             o0, o1, o2, o3):
        eh = e_ref[...]
        ssum = jnp.concatenate(
            [s0[...], s1[...], s2[...], s3[...]], axis=1)
        cnt = c0_ref[...][:, :1] + c1_ref[...][:, :1]
        mean = ssum / jnp.maximum(cnt, 1.0)
        h2 = jnp.maximum(
            eh + mean + bl_ref[...]
            + jnp.dot(eh, wr_ref[...], preferred_element_type=jnp.float32), 0.0)
        y = jnp.dot(h2, wn_ref[...], preferred_element_type=jnp.float32)
        for k, o in enumerate((o0, o1, o2, o3)):
            o[...] = y[:, k * _S:(k + 1) * _S]

    return pl.pallas_call(
        body,
        grid=(n // _BLK,),
        in_specs=[pl.BlockSpec((_BLK, _H), lambda i: (i, 0))]
                 + _shard_specs(6)
                 + [pl.BlockSpec((_H, _H), lambda i: (0, 0)),
                    pl.BlockSpec((1, _H), lambda i: (0, 0)),
                    pl.BlockSpec((_H, _H), lambda i: (0, 0))],
        out_specs=_shard_specs(4),
        out_shape=_shard_outs(n),
    )(evt_h, *s4, c0, c1, wr, bl.reshape(1, _H), wl_next)


def _loc_head(loc_x, s4, c0, c1, w_loc, b_loc, wr, bl, w1, b1, w2, b2):
    n = loc_x.shape[0]
    hh = _H // 2

    def body(x_ref, s0, s1, s2, s3, c0_ref, c1_ref, wloc_ref, bloc_ref,
             wr_ref, bl_ref, w1_ref, b1_ref, w2_ref, b2_ref, o_ref):
        lh = jnp.maximum(
            jnp.dot(x_ref[...], wloc_ref[...], preferred_element_type=jnp.float32)
            + bloc_ref[...], 0.0)
        ssum = jnp.concatenate(
            [s0[...], s1[...], s2[...], s3[...]], axis=1)
        cnt = c0_ref[...][:, :1] + c1_ref[...][:, :1]
        mean = ssum / jnp.maximum(cnt, 1.0)
        h2 = jnp.maximum(
            mean + bl_ref[...]
            + jnp.dot(lh, wr_ref[...], preferred_element_type=jnp.float32), 0.0)
        hid = jnp.maximum(
            jnp.dot(h2, w1_ref[...], preferred_element_type=jnp.float32)
            + b1_ref[...], 0.0)
        o_ref[...] = (jnp.dot(hid, w2_ref[...], preferred_element_type=jnp.float32)
                      + b2_ref[...])

    return pl.pallas_call(
        body,
        grid=(n // _BLK,),
        in_specs=[pl.BlockSpec((_BLK, _H), lambda i: (i, 0))]
                 + _shard_specs(6)
                 + [pl.BlockSpec((_H, _H), lambda i: (0, 0)),
                    pl.BlockSpec((1, _H), lambda i: (0, 0)),
                    pl.BlockSpec((_H, _H), lambda i: (0, 0)),
                    pl.BlockSpec((1, _H), lambda i: (0, 0)),
                    pl.BlockSpec((_H, hh), lambda i: (0, 0)),
                    pl.BlockSpec((1, hh), lambda i: (0, 0)),
                    pl.BlockSpec((hh, 1), lambda i: (0, 0)),
                    pl.BlockSpec((1, 1), lambda i: (0, 0))],
        out_specs=pl.BlockSpec((_BLK, 1), lambda i: (i, 0)),
        out_shape=jax.ShapeDtypeStruct((n, 1), jnp.float32),
    )(loc_x, *s4, c0, c1, w_loc, b_loc.reshape(1, _H), wr, bl.reshape(1, _H),
      w1, b1.reshape(1, hh), w2, b2.reshape(1, 1))


def kernel(location_x, event_x, actor_x, actor_edge_index, event_edge_index,
           W_loc, b_loc, W_evt, b_evt, W_act, b_act, Wl_ae, bl_ae, Wr_ae,
           Wl_el, bl_el, Wr_el, W1, b1, W2, b2):
    n_evt = event_x.shape[0]
    n_loc = location_x.shape[0]

    act_y = _enc2(actor_x, W_act, b_act, Wl_ae)
    evt_h = _enc(event_x, W_evt, b_evt)

    *s_ae, ca0, ca1 = _seg_sum_sc(act_y, actor_edge_index[0],
                                  actor_edge_index[1], n_dst=n_evt)

    evt_z = _evt_update(evt_h, s_ae, ca0, ca1, Wr_ae, bl_ae, Wl_el)

    *s_el, ce0, ce1 = _seg_sum_sc(evt_z, event_edge_index[0],
                                  event_edge_index[1], n_dst=n_loc)

    out = _loc_head(location_x, s_el, ce0, ce1, W_loc, b_loc, Wr_el, bl_el,
                    W1, b1, W2, b2)
    return out[:, 0]


# speedup vs baseline: 1.0720x; 1.0720x over previous
import jax
import jax.numpy as jnp
from jax import lax
from jax.experimental import pallas as pl
from jax.experimental.pallas import tpu as pltpu
from jax.experimental.pallas import tpu_sc as plsc

_NC = 2
_NS = 16
_B = 128
_RPC = 32
_NBUF = 4
_H = 128
_S = 32


def _seg_sum_sc(xs, src, dst, n_dst):
    e = src.shape[0]
    ept = ((e + _NS * _RPC * _B - 1) // (_NS * _RPC * _B)) * _RPC * _B
    e_pad = ept * _NS
    rows_pt = ept // _B
    chunks = rows_pt // _RPC
    half_rows_pt = rows_pt // 2
    half_chunks = chunks // 2
    n_dst_pad = ((n_dst + 1 + 63) // 64) * 64
    zc = n_dst_pad // 32
    z_iter = (zc + _NS - 1) // _NS

    if e_pad > e:
        pad = e_pad - e
        src = jnp.concatenate([src, jnp.zeros((pad,), jnp.int32)])
        dst = jnp.concatenate([dst, jnp.full((pad,), jnp.int32(n_dst))])
    src2 = src.reshape(e_pad // _B, _B)
    dst2 = dst.reshape(e_pad // _B, _B)

    mesh = plsc.VectorSubcoreMesh(core_axis_name="c", subcore_axis_name="s")

    def body(x0, x1, x2, x3, src_hbm, dst_hbm, o0, o1, o2, o3, c0, c1,
             src_st, dst_st, rows, ones_blk, zrow, acc, sem, sem2):
        cid = lax.axis_index("c")
        sid = lax.axis_index("s")
        lane = lax.iota(jnp.int32, 16)
        onev = (1 - jnp.minimum(lane, 1)).astype(jnp.float32)

        def _zr(r, carry):
            for q in range(_S // 16):
                zrow[r, pl.ds(q * 16, 16)] = jnp.zeros((16,), jnp.float32)
            return carry
        lax.fori_loop(0, 32, _zr, 0)

        def _ob(r, carry):
            ones_blk[r, pl.ds(0, 16)] = onev
            ones_blk[r, pl.ds(16, 16)] = jnp.zeros((16,), jnp.float32)
            return carry
        lax.fori_loop(0, _B, _ob, 0)

        def zero_acc():
            def zb(z, carry):
                czk = z * _NS + sid
                @pl.when(czk < zc)
                def _():
                    pltpu.sync_copy(zrow, acc.at[pl.ds(czk * 32, 32), :])
                return carry
            lax.fori_loop(0, z_iter, zb, 0)

        def write_out(o):
            def wb(w, carry):
                wck = w * _NS + sid
                @pl.when(wck < zc)
                def _():
                    pltpu.sync_copy(acc.at[pl.ds(wck * 32, 32), :],
                                    o.at[pl.ds(wck * 32, 32), :])
                return carry
            lax.fori_loop(0, z_iter, wb, 0)

        def scan_pass(x, row0, nchunks):
            ngroups = _RPC

            def gather_desc(gi, par):
                return pltpu.make_async_copy(
                    x.at[src_st.at[gi]], rows.at[par], sem)

            def chunk_body(ch, carry):
                base = row0 + ch * _RPC
                pltpu.sync_copy(src_hbm.at[pl.ds(base, _RPC), :], src_st)
                pltpu.sync_copy(dst_hbm.at[pl.ds(base, _RPC), :], dst_st)
                for pre in range(_NBUF - 1):
                    gather_desc(pre, pre).start()

                def grp(g, carry2):
                    par = g & (_NBUF - 1)
                    @pl.when(g + _NBUF - 1 < ngroups)
                    def _():
                        gather_desc(g + _NBUF - 1,
                                    (g + _NBUF - 1) & (_NBUF - 1)).start()
                    gather_desc(g, par).wait()
                    pltpu.sync_copy(rows.at[par],
                                    acc.at[dst_st.at[g]], add=True)
                    return carry2
                lax.fori_loop(0, ngroups, grp, 0)
                return carry
            lax.fori_loop(0, nchunks, chunk_body, 0)

        def count_pass(row0, nchunks):
            def chunk_body(ch, carry):
                base = row0 + ch * _RPC
                pltpu.sync_copy(dst_hbm.at[pl.ds(base, _RPC), :], dst_st)

                def one_row(j, carry2):
                    pltpu.async_copy(ones_blk, acc.at[dst_st.at[j]], sem2,
                                     add=True)
                    return carry2
                lax.fori_loop(0, _RPC, one_row, 0)

                def drain(j, carry2):
                    pltpu.make_async_copy(ones_blk,
                                          acc.at[dst_st.at[j]], sem2).wait()
                    return carry2
                lax.fori_loop(0, _RPC, drain, 0)
                return carry
            lax.fori_loop(0, nchunks, chunk_body, 0)

        xs_refs = (x0, x1, x2, x3)
        os_refs = (o0, o1, o2, o3)
        for p in range(4):
            own = cid == (p // 2)
            @pl.when(own)
            def _():
                zero_acc()
            plsc.subcore_barrier()
            @pl.when(own)
            def _():
                scan_pass(xs_refs[p], sid * rows_pt, chunks)
            plsc.subcore_barrier()
            @pl.when(own)
            def _():
                write_out(os_refs[p])
            plsc.subcore_barrier()

        zero_acc()
        plsc.subcore_barrier()
        count_pass(cid * (e_pad // _B // 2) + sid * half_rows_pt, half_chunks)
        plsc.subcore_barrier()
        @pl.when(cid == 0)
        def _():
            write_out(c0)
        @pl.when(cid == 1)
        def _():
            write_out(c1)

    kern = pl.kernel(
        body,
        out_type=tuple(jax.ShapeDtypeStruct((n_dst_pad, _S), jnp.float32)
                       for _ in range(6)),
        mesh=mesh,
        compiler_params=pltpu.CompilerParams(use_tc_tiling_on_sc=False),
        scratch_types=[
            pltpu.VMEM((_RPC, _B), jnp.int32),
            pltpu.VMEM((_RPC, _B), jnp.int32),
            pltpu.VMEM((_NBUF, _B, _S), jnp.float32),
            pltpu.VMEM((_B, _S), jnp.float32),
            pltpu.VMEM((32, _S), jnp.float32),
            pltpu.VMEM_SHARED((n_dst_pad, _S), jnp.float32),
            pltpu.SemaphoreType.DMA,
            pltpu.SemaphoreType.DMA,
        ],
    )
    return kern(xs[0], xs[1], xs[2], xs[3], src2, dst2)


_BLK = 2000


def _shard_specs(n):
    return [pl.BlockSpec((_BLK, _S), lambda i: (i, 0)) for _ in range(n)]


def _shard_outs(n):
    return tuple(jax.ShapeDtypeStruct((n, _S), jnp.float32) for _ in range(4))


def _enc2(x, w, b, w2):
    n = x.shape[0]

    def body(x_ref, w_ref, b_ref, w2_ref, o0, o1, o2, o3):
        h = jnp.maximum(jnp.dot(x_ref[...], w_ref[...],
                                preferred_element_type=jnp.float32)
                        + b_ref[...], 0.0)
        y = jnp.dot(h, w2_ref[...], preferred_element_type=jnp.float32)
        for k, o in enumerate((o0, o1, o2, o3)):
            o[...] = y[:, k * _S:(k + 1) * _S]

    return pl.pallas_call(
        body,
        grid=(n // _BLK,),
        in_specs=[pl.BlockSpec((_BLK, _H), lambda i: (i, 0)),
                  pl.BlockSpec((_H, _H), lambda i: (0, 0)),
                  pl.BlockSpec((1, _H), lambda i: (0, 0)),
                  pl.BlockSpec((_H, _H), lambda i: (0, 0))],
        out_specs=_shard_specs(4),
        out_shape=_shard_outs(n),
    )(x, w, b.reshape(1, _H), w2)


def _enc(x, w, b):
    n = x.shape[0]

    def body(x_ref, w_ref, b_ref, o_ref):
        o_ref[...] = jnp.maximum(
            jnp.dot(x_ref[...], w_ref[...], preferred_element_type=jnp.float32)
            + b_ref[...], 0.0)

    return pl.pallas_call(
        body,
        grid=(n // _BLK,),
        in_specs=[pl.BlockSpec((_BLK, _H), lambda i: (i, 0)),
                  pl.BlockSpec((_H, _H), lambda i: (0, 0)),
                  pl.BlockSpec((1, _H), lambda i: (0, 0))],
        out_specs=pl.BlockSpec((_BLK, _H), lambda i: (i, 0)),
        out_shape=jax.ShapeDtypeStruct((n, _H), jnp.float32),
    )(x, w, b.reshape(1, _H))


def _evt_update(evt_h, s4, c0, c1, wr, bl, wl_next):
    n = evt_h.shape[0]

    def body(e_ref, s0, s1, s2, s3, c0_ref, c1_ref, wr_ref, bl_ref, wn_ref,
             o0, o1, o2, o3):
        eh = e_ref[...]
        ssum = jnp.concatenate(
            [s0[...], s1[...], s2[...], s3[...]], axis=1)
        cnt = c0_ref[...][:, :1] + c1_ref[...][:, :1]
        mean = ssum / jnp.maximum(cnt, 1.0)
        h2 = jnp.maximum(
            eh + mean + bl_ref[...]
            + jnp.dot(eh, wr_ref[...], preferred_element_type=jnp.float32), 0.0)
        y = jnp.dot(h2, wn_ref[...], preferred_element_type=jnp.float32)
        for k, o in enumerate((o0, o1, o2, o3)):
            o[...] = y[:, k * _S:(k + 1) * _S]

    return pl.pallas_call(
        body,
        grid=(n // _BLK,),
        in_specs=[pl.BlockSpec((_BLK, _H), lambda i: (i, 0))]
                 + _shard_specs(6)
                 + [pl.BlockSpec((_H, _H), lambda i: (0, 0)),
                    pl.BlockSpec((1, _H), lambda i: (0, 0)),
                    pl.BlockSpec((_H, _H), lambda i: (0, 0))],
        out_specs=_shard_specs(4),
        out_shape=_shard_outs(n),
    )(evt_h, *s4, c0, c1, wr, bl.reshape(1, _H), wl_next)


def _loc_head(loc_x, s4, c0, c1, w_loc, b_loc, wr, bl, w1, b1, w2, b2):
    n = loc_x.shape[0]
    hh = _H // 2

    def body(x_ref, s0, s1, s2, s3, c0_ref, c1_ref, wloc_ref, bloc_ref,
             wr_ref, bl_ref, w1_ref, b1_ref, w2_ref, b2_ref, o_ref):
        lh = jnp.maximum(
            jnp.dot(x_ref[...], wloc_ref[...], preferred_element_type=jnp.float32)
            + bloc_ref[...], 0.0)
        ssum = jnp.concatenate(
            [s0[...], s1[...], s2[...], s3[...]], axis=1)
        cnt = c0_ref[...][:, :1] + c1_ref[...][:, :1]
        mean = ssum / jnp.maximum(cnt, 1.0)
        h2 = jnp.maximum(
            mean + bl_ref[...]
            + jnp.dot(lh, wr_ref[...], preferred_element_type=jnp.float32), 0.0)
        hid = jnp.maximum(
            jnp.dot(h2, w1_ref[...], preferred_element_type=jnp.float32)
            + b1_ref[...], 0.0)
        o_ref[...] = (jnp.dot(hid, w2_ref[...], preferred_element_type=jnp.float32)
                      + b2_ref[...])

    return pl.pallas_call(
        body,
        grid=(n // _BLK,),
        in_specs=[pl.BlockSpec((_BLK, _H), lambda i: (i, 0))]
                 + _shard_specs(6)
                 + [pl.BlockSpec((_H, _H), lambda i: (0, 0)),
                    pl.BlockSpec((1, _H), lambda i: (0, 0)),
                    pl.BlockSpec((_H, _H), lambda i: (0, 0)),
                    pl.BlockSpec((1, _H), lambda i: (0, 0)),
                    pl.BlockSpec((_H, hh), lambda i: (0, 0)),
                    pl.BlockSpec((1, hh), lambda i: (0, 0)),
                    pl.BlockSpec((hh, 1), lambda i: (0, 0)),
                    pl.BlockSpec((1, 1), lambda i: (0, 0))],
        out_specs=pl.BlockSpec((_BLK, 1), lambda i: (i, 0)),
        out_shape=jax.ShapeDtypeStruct((n, 1), jnp.float32),
    )(loc_x, *s4, c0, c1, w_loc, b_loc.reshape(1, _H), wr, bl.reshape(1, _H),
      w1, b1.reshape(1, hh), w2, b2.reshape(1, 1))


def kernel(location_x, event_x, actor_x, actor_edge_index, event_edge_index,
           W_loc, b_loc, W_evt, b_evt, W_act, b_act, Wl_ae, bl_ae, Wr_ae,
           Wl_el, bl_el, Wr_el, W1, b1, W2, b2):
    n_evt = event_x.shape[0]
    n_loc = location_x.shape[0]

    act_y = _enc2(actor_x, W_act, b_act, Wl_ae)
    evt_h = _enc(event_x, W_evt, b_evt)

    *s_ae, ca0, ca1 = _seg_sum_sc(act_y, actor_edge_index[0],
                                  actor_edge_index[1], n_dst=n_evt)

    evt_z = _evt_update(evt_h, s_ae, ca0, ca1, Wr_ae, bl_ae, Wl_el)

    *s_el, ce0, ce1 = _seg_sum_sc(evt_z, event_edge_index[0],
                                  event_edge_index[1], n_dst=n_loc)

    out = _loc_head(location_x, s_el, ce0, ce1, W_loc, b_loc, Wr_el, bl_el,
                    W1, b1, W2, b2)
    return out[:, 0]


# speedup vs baseline: 1.0737x; 1.0016x over previous
import jax
import jax.numpy as jnp
from jax import lax
from jax.experimental import pallas as pl
from jax.experimental.pallas import tpu as pltpu
from jax.experimental.pallas import tpu_sc as plsc

_NC = 2
_NS = 16
_B = 128
_RPC = 32
_NBUF = 4
_H = 128
_S = 32


def _seg_sum_sc(xs, src, dst, n_dst):
    e = src.shape[0]
    ept = ((e + _NS * _RPC * _B - 1) // (_NS * _RPC * _B)) * _RPC * _B
    e_pad = ept * _NS
    rows_pt = ept // _B
    chunks = rows_pt // _RPC
    half_rows_pt = rows_pt // 2
    half_chunks = chunks // 2
    n_dst_pad = ((n_dst + 1 + 63) // 64) * 64
    zc = n_dst_pad // 32
    z_iter = (zc + _NS - 1) // _NS

    if e_pad > e:
        pad = e_pad - e
        src = jnp.concatenate([src, jnp.zeros((pad,), jnp.int32)])
        dst = jnp.concatenate([dst, jnp.full((pad,), jnp.int32(n_dst))])
    src2 = src.reshape(e_pad // _B, _B)
    dst2 = dst.reshape(e_pad // _B, _B)

    mesh = plsc.VectorSubcoreMesh(core_axis_name="c", subcore_axis_name="s")

    def body(x0, x1, x2, x3, src_hbm, dst_hbm, o0, o1, o2, o3, c0, c1,
             src_st, dst_st, rows, ones_blk, zrow, acc, sem, sem2):
        cid = lax.axis_index("c")
        sid = lax.axis_index("s")
        lane = lax.iota(jnp.int32, 16)
        onev = (1 - jnp.minimum(lane, 1)).astype(jnp.float32)

        def _zr(r, carry):
            for q in range(_S // 16):
                zrow[r, pl.ds(q * 16, 16)] = jnp.zeros((16,), jnp.float32)
            return carry
        lax.fori_loop(0, 32, _zr, 0)

        def _ob(r, carry):
            ones_blk[r, pl.ds(0, 16)] = onev
            ones_blk[r, pl.ds(16, 16)] = jnp.zeros((16,), jnp.float32)
            return carry
        lax.fori_loop(0, _B, _ob, 0)

        def zero_acc():
            def zb(z, carry):
                czk = z * _NS + sid
                @pl.when(czk < zc)
                def _():
                    pltpu.sync_copy(zrow, acc.at[pl.ds(czk * 32, 32), :])
                return carry
            lax.fori_loop(0, z_iter, zb, 0)

        def write_out(o):
            def wb(w, carry):
                wck = w * _NS + sid
                @pl.when(wck < zc)
                def _():
                    pltpu.sync_copy(acc.at[pl.ds(wck * 32, 32), :],
                                    o.at[pl.ds(wck * 32, 32), :])
                return carry
            lax.fori_loop(0, z_iter, wb, 0)

        def scan_pass(x, row0, nchunks):
            ngroups = _RPC

            def gather_desc(gi, par):
                return pltpu.make_async_copy(
                    x.at[src_st.at[gi]], rows.at[par], sem)

            def chunk_body(ch, carry):
                base = row0 + ch * _RPC
                pltpu.sync_copy(src_hbm.at[pl.ds(base, _RPC), :], src_st)
                pltpu.sync_copy(dst_hbm.at[pl.ds(base, _RPC), :], dst_st)
                for pre in range(_NBUF - 1):
                    gather_desc(pre, pre).start()

                def scat_desc(gi, par):
                    return pltpu.make_async_copy(
                        rows.at[par], acc.at[dst_st.at[gi]], sem2)

                def grp(g, carry2):
                    par = g & (_NBUF - 1)
                    @pl.when(g > 0)
                    def _():
                        scat_desc(g - 1, (g - 1) & (_NBUF - 1)).wait()
                    @pl.when(g + _NBUF - 1 < ngroups)
                    def _():
                        gather_desc(g + _NBUF - 1,
                                    (g + _NBUF - 1) & (_NBUF - 1)).start()
                    gather_desc(g, par).wait()
                    pltpu.async_copy(rows.at[par], acc.at[dst_st.at[g]],
                                     sem2, add=True)
                    return carry2
                lax.fori_loop(0, ngroups, grp, 0)
                scat_desc(ngroups - 1, (ngroups - 1) & (_NBUF - 1)).wait()
                return carry
            lax.fori_loop(0, nchunks, chunk_body, 0)

        def count_pass(row0, nchunks):
            def chunk_body(ch, carry):
                base = row0 + ch * _RPC
                pltpu.sync_copy(dst_hbm.at[pl.ds(base, _RPC), :], dst_st)

                def one_row(j, carry2):
                    pltpu.async_copy(ones_blk, acc.at[dst_st.at[j]], sem2,
                                     add=True)
                    return carry2
                lax.fori_loop(0, _RPC, one_row, 0)

                def drain(j, carry2):
                    pltpu.make_async_copy(ones_blk,
                                          acc.at[dst_st.at[j]], sem2).wait()
                    return carry2
                lax.fori_loop(0, _RPC, drain, 0)
                return carry
            lax.fori_loop(0, nchunks, chunk_body, 0)

        xs_refs = (x0, x1, x2, x3)
        os_refs = (o0, o1, o2, o3)
        for p in range(4):
            own = cid == (p // 2)
            @pl.when(own)
            def _():
                zero_acc()
            plsc.subcore_barrier()
            @pl.when(own)
            def _():
                scan_pass(xs_refs[p], sid * rows_pt, chunks)
            plsc.subcore_barrier()
            @pl.when(own)
            def _():
                write_out(os_refs[p])
            plsc.subcore_barrier()

        zero_acc()
        plsc.subcore_barrier()
        count_pass(cid * (e_pad // _B // 2) + sid * half_rows_pt, half_chunks)
        plsc.subcore_barrier()
        @pl.when(cid == 0)
        def _():
            write_out(c0)
        @pl.when(cid == 1)
        def _():
            write_out(c1)

    kern = pl.kernel(
        body,
        out_type=tuple(jax.ShapeDtypeStruct((n_dst_pad, _S), jnp.float32)
                       for _ in range(6)),
        mesh=mesh,
        compiler_params=pltpu.CompilerParams(use_tc_tiling_on_sc=False),
        scratch_types=[
            pltpu.VMEM((_RPC, _B), jnp.int32),
            pltpu.VMEM((_RPC, _B), jnp.int32),
            pltpu.VMEM((_NBUF, _B, _S), jnp.float32),
            pltpu.VMEM((_B, _S), jnp.float32),
            pltpu.VMEM((32, _S), jnp.float32),
            pltpu.VMEM_SHARED((n_dst_pad, _S), jnp.float32),
            pltpu.SemaphoreType.DMA,
            pltpu.SemaphoreType.DMA,
        ],
    )
    return kern(xs[0], xs[1], xs[2], xs[3], src2, dst2)


_BLK = 2000


def _shard_specs(n):
    return [pl.BlockSpec((_BLK, _S), lambda i: (i, 0)) for _ in range(n)]


def _shard_outs(n):
    return tuple(jax.ShapeDtypeStruct((n, _S), jnp.float32) for _ in range(4))


def _enc2(x, w, b, w2):
    n = x.shape[0]

    def body(x_ref, w_ref, b_ref, w2_ref, o0, o1, o2, o3):
        h = jnp.maximum(jnp.dot(x_ref[...], w_ref[...],
                                preferred_element_type=jnp.float32)
                        + b_ref[...], 0.0)
        y = jnp.dot(h, w2_ref[...], preferred_element_type=jnp.float32)
        for k, o in enumerate((o0, o1, o2, o3)):
            o[...] = y[:, k * _S:(k + 1) * _S]

    return pl.pallas_call(
        body,
        grid=(n // _BLK,),
        in_specs=[pl.BlockSpec((_BLK, _H), lambda i: (i, 0)),
                  pl.BlockSpec((_H, _H), lambda i: (0, 0)),
                  pl.BlockSpec((1, _H), lambda i: (0, 0)),
                  pl.BlockSpec((_H, _H), lambda i: (0, 0))],
        out_specs=_shard_specs(4),
        out_shape=_shard_outs(n),
    )(x, w, b.reshape(1, _H), w2)


def _enc(x, w, b):
    n = x.shape[0]

    def body(x_ref, w_ref, b_ref, o_ref):
        o_ref[...] = jnp.maximum(
            jnp.dot(x_ref[...], w_ref[...], preferred_element_type=jnp.float32)
            + b_ref[...], 0.0)

    return pl.pallas_call(
        body,
        grid=(n // _BLK,),
        in_specs=[pl.BlockSpec((_BLK, _H), lambda i: (i, 0)),
                  pl.BlockSpec((_H, _H), lambda i: (0, 0)),
                  pl.BlockSpec((1, _H), lambda i: (0, 0))],
        out_specs=pl.BlockSpec((_BLK, _H), lambda i: (i, 0)),
        out_shape=jax.ShapeDtypeStruct((n, _H), jnp.float32),
    )(x, w, b.reshape(1, _H))


def _evt_update(evt_h, s4, c0, c1, wr, bl, wl_next):
    n = evt_h.shape[0]

    def body(e_ref, s0, s1, s2, s3, c0_ref, c1_ref, wr_ref, bl_ref, wn_ref,
             o0, o1, o2, o3):
        eh = e_ref[...]
        ssum = jnp.concatenate(
            [s0[...], s1[...], s2[...], s3[...]], axis=1)
        cnt = c0_ref[...][:, :1] + c1_ref[...][:, :1]
        mean = ssum / jnp.maximum(cnt, 1.0)
        h2 = jnp.maximum(
            eh + mean + bl_ref[...]
            + jnp.dot(eh, wr_ref[...], preferred_element_type=jnp.float32), 0.0)
        y = jnp.dot(h2, wn_ref[...], preferred_element_type=jnp.float32)
        for k, o in enumerate((o0, o1, o2, o3)):
            o[...] = y[:, k * _S:(k + 1) * _S]

    return pl.pallas_call(
        body,
        grid=(n // _BLK,),
        in_specs=[pl.BlockSpec((_BLK, _H), lambda i: (i, 0))]
                 + _shard_specs(6)
                 + [pl.BlockSpec((_H, _H), lambda i: (0, 0)),
                    pl.BlockSpec((1, _H), lambda i: (0, 0)),
                    pl.BlockSpec((_H, _H), lambda i: (0, 0))],
        out_specs=_shard_specs(4),
        out_shape=_shard_outs(n),
    )(evt_h, *s4, c0, c1, wr, bl.reshape(1, _H), wl_next)


def _loc_head(loc_x, s4, c0, c1, w_loc, b_loc, wr, bl, w1, b1, w2, b2):
    n = loc_x.shape[0]
    hh = _H // 2

    def body(x_ref, s0, s1, s2, s3, c0_ref, c1_ref, wloc_ref, bloc_ref,
             wr_ref, bl_ref, w1_ref, b1_ref, w2_ref, b2_ref, o_ref):
        lh = jnp.maximum(
            jnp.dot(x_ref[...], wloc_ref[...], preferred_element_type=jnp.float32)
            + bloc_ref[...], 0.0)
        ssum = jnp.concatenate(
            [s0[...], s1[...], s2[...], s3[...]], axis=1)
        cnt = c0_ref[...][:, :1] + c1_ref[...][:, :1]
        mean = ssum / jnp.maximum(cnt, 1.0)
        h2 = jnp.maximum(
            mean + bl_ref[...]
            + jnp.dot(lh, wr_ref[...], preferred_element_type=jnp.float32), 0.0)
        hid = jnp.maximum(
            jnp.dot(h2, w1_ref[...], preferred_element_type=jnp.float32)
            + b1_ref[...], 0.0)
        o_ref[...] = (jnp.dot(hid, w2_ref[...], preferred_element_type=jnp.float32)
                      + b2_ref[...])

    return pl.pallas_call(
        body,
        grid=(n // _BLK,),
        in_specs=[pl.BlockSpec((_BLK, _H), lambda i: (i, 0))]
                 + _shard_specs(6)
                 + [pl.BlockSpec((_H, _H), lambda i: (0, 0)),
                    pl.BlockSpec((1, _H), lambda i: (0, 0)),
                    pl.BlockSpec((_H, _H), lambda i: (0, 0)),
                    pl.BlockSpec((1, _H), lambda i: (0, 0)),
                    pl.BlockSpec((_H, hh), lambda i: (0, 0)),
                    pl.BlockSpec((1, hh), lambda i: (0, 0)),
                    pl.BlockSpec((hh, 1), lambda i: (0, 0)),
                    pl.BlockSpec((1, 1), lambda i: (0, 0))],
        out_specs=pl.BlockSpec((_BLK, 1), lambda i: (i, 0)),
        out_shape=jax.ShapeDtypeStruct((n, 1), jnp.float32),
    )(loc_x, *s4, c0, c1, w_loc, b_loc.reshape(1, _H), wr, bl.reshape(1, _H),
      w1, b1.reshape(1, hh), w2, b2.reshape(1, 1))


def kernel(location_x, event_x, actor_x, actor_edge_index, event_edge_index,
           W_loc, b_loc, W_evt, b_evt, W_act, b_act, Wl_ae, bl_ae, Wr_ae,
           Wl_el, bl_el, Wr_el, W1, b1, W2, b2):
    n_evt = event_x.shape[0]
    n_loc = location_x.shape[0]

    act_y = _enc2(actor_x, W_act, b_act, Wl_ae)
    evt_h = _enc(event_x, W_evt, b_evt)

    *s_ae, ca0, ca1 = _seg_sum_sc(act_y, actor_edge_index[0],
                                  actor_edge_index[1], n_dst=n_evt)

    evt_z = _evt_update(evt_h, s_ae, ca0, ca1, Wr_ae, bl_ae, Wl_el)

    *s_el, ce0, ce1 = _seg_sum_sc(evt_z, event_edge_index[0],
                                  event_edge_index[1], n_dst=n_loc)

    out = _loc_head(location_x, s_el, ce0, ce1, W_loc, b_loc, Wr_el, bl_el,
                    W1, b1, W2, b2)
    return out[:, 0]
